# baseline (speedup 1.0000x reference)
import jax
import jax.numpy as jnp
from jax import lax
from jax.experimental import pallas as pl
from jax.experimental.pallas import tpu as pltpu
from jax.experimental.pallas import tpu_sc as plsc

_B = 4
_L = 8192
_D = 1024
_NC = 2
_NS = 16
_NW = _NC * _NS
_RPW = _L // _NW
_C = 8
_NCH = _RPW // _C
_CW = _C * _D


def _sc_body(x_hbm, e_hbm, o_hbm, xb, eb, sin0, sin1, sout0, sout1):
    wid = lax.axis_index("s") * _NC + lax.axis_index("c")
    base = wid * _RPW * _D
    sin = (sin0, sin1)
    sout = (sout0, sout1)

    def off(c):
        return pl.multiple_of(base + c * _CW, _CW)

    def fire_in(s, c):
        o = off(c)
        pltpu.async_copy(e_hbm.at[pl.ds(o, _CW)], eb.at[s], sin[s])
        for b in range(_B):
            pltpu.async_copy(x_hbm.at[b, pl.ds(o, _CW)], xb.at[s, b], sin[s])

    def wait_in(s, c):
        o = off(c)
        pltpu.make_async_copy(e_hbm.at[pl.ds(o, _CW)], eb.at[s], sin[s]).wait()
        for b in range(_B):
            pltpu.make_async_copy(
                x_hbm.at[b, pl.ds(o, _CW)], xb.at[s, b], sin[s]
            ).wait()

    def fire_out(s, c):
        o = off(c)
        for b in range(_B):
            pltpu.async_copy(xb.at[s, b], o_hbm.at[b, pl.ds(o, _CW)], sout[s])

    def wait_out(s, c):
        o = off(c)
        for b in range(_B):
            pltpu.make_async_copy(
                xb.at[s, b], o_hbm.at[b, pl.ds(o, _CW)], sout[s]
            ).wait()

    def compute(s):
        er = eb.at[s]
        xr = [xb.at[s, b] for b in range(_B)]

        def step(i, carry):
            sl = pl.ds(i * 16, 16)
            v = er[sl]
            for b in range(_B):
                xr[b][sl] = xr[b][sl] + v
            return carry

        lax.fori_loop(0, _CW // 16, step, 0)

    fire_in(0, 0)
    fire_in(1, 1)

    def body(k, carry):
        cc = 2 * k
        wait_in(0, cc)
        compute(0)
        fire_out(0, cc)
        wait_in(1, cc + 1)
        compute(1)
        fire_out(1, cc + 1)
        wait_out(0, cc)

        @pl.when(cc + 2 < _NCH)
        def _():
            fire_in(0, cc + 2)

        wait_out(1, cc + 1)

        @pl.when(cc + 3 < _NCH)
        def _():
            fire_in(1, cc + 3)

        return carry

    lax.fori_loop(0, _NCH // 2, body, 0)


def kernel(input, emb):
    x2 = input.reshape(_B, _L * _D)
    e2 = emb.reshape(_L * _D)
    run = pl.kernel(
        _sc_body,
        out_type=jax.ShapeDtypeStruct((_B, _L * _D), jnp.float32),
        mesh=plsc.VectorSubcoreMesh(core_axis_name="c", subcore_axis_name="s"),
        scratch_types=[
            pltpu.VMEM((2, _B, _CW), jnp.float32),
            pltpu.VMEM((2, _CW), jnp.float32),
            pltpu.SemaphoreType.DMA,
            pltpu.SemaphoreType.DMA,
            pltpu.SemaphoreType.DMA,
            pltpu.SemaphoreType.DMA,
        ],
    )
    return run(x2, e2).reshape(_B, _L, _D)


# speedup vs baseline: 1.1436x; 1.1436x over previous
import jax
import jax.numpy as jnp
from jax import lax
from jax.experimental import pallas as pl
from jax.experimental.pallas import tpu as pltpu
from jax.experimental.pallas import tpu_sc as plsc

_B = 4
_L = 8192
_D = 1024
_NC = 2
_NS = 16
_NW = _NC * _NS
_RPW = _L // _NW
_C = 8
_NCH = _RPW // _C
_CW = _C * _D


def _sc_body(x_hbm, e_hbm, o_hbm, xb, eb, sin0, sin1, sout0, sout1):
    wid = lax.axis_index("s") * _NC + lax.axis_index("c")
    base = wid * _RPW * _D
    sin = (sin0, sin1)
    sout = (sout0, sout1)

    def off(c):
        return pl.multiple_of(base + c * _CW, _CW)

    def fire_in(s, c):
        o = off(c)
        pltpu.async_copy(e_hbm.at[pl.ds(o, _CW)], eb.at[s], sin[s])
        for b in range(_B):
            pltpu.async_copy(x_hbm.at[b, pl.ds(o, _CW)], xb.at[s, b], sin[s])

    def wait_in(s, c):
        o = off(c)
        pltpu.make_async_copy(e_hbm.at[pl.ds(o, _CW)], eb.at[s], sin[s]).wait()
        for b in range(_B):
            pltpu.make_async_copy(
                x_hbm.at[b, pl.ds(o, _CW)], xb.at[s, b], sin[s]
            ).wait()

    def fire_out(s, c):
        o = off(c)
        for b in range(_B):
            pltpu.async_copy(xb.at[s, b], o_hbm.at[b, pl.ds(o, _CW)], sout[s])

    def wait_out(s, c):
        o = off(c)
        for b in range(_B):
            pltpu.make_async_copy(
                xb.at[s, b], o_hbm.at[b, pl.ds(o, _CW)], sout[s]
            ).wait()

    def compute(s):
        er = eb.at[s]
        xr = [xb.at[s, b] for b in range(_B)]

        @plsc.parallel_loop(0, _CW, step=16, unroll=8)
        def step(i):
            sl = pl.ds(i, 16)
            v = er[sl]
            for b in range(_B):
                xr[b][sl] = xr[b][sl] + v

    fire_in(0, 0)
    fire_in(1, 1)

    def body(k, carry):
        cc = 2 * k
        wait_in(0, cc)
        compute(0)
        fire_out(0, cc)
        wait_in(1, cc + 1)
        compute(1)
        fire_out(1, cc + 1)
        wait_out(0, cc)

        @pl.when(cc + 2 < _NCH)
        def _():
            fire_in(0, cc + 2)

        wait_out(1, cc + 1)

        @pl.when(cc + 3 < _NCH)
        def _():
            fire_in(1, cc + 3)

        return carry

    lax.fori_loop(0, _NCH // 2, body, 0)


def kernel(input, emb):
    x2 = input.reshape(_B, _L * _D)
    e2 = emb.reshape(_L * _D)
    run = pl.kernel(
        _sc_body,
        out_type=jax.ShapeDtypeStruct((_B, _L * _D), jnp.float32),
        mesh=plsc.VectorSubcoreMesh(core_axis_name="c", subcore_axis_name="s"),
        scratch_types=[
            pltpu.VMEM((2, _B, _CW), jnp.float32),
            pltpu.VMEM((2, _CW), jnp.float32),
            pltpu.SemaphoreType.DMA,
            pltpu.SemaphoreType.DMA,
            pltpu.SemaphoreType.DMA,
            pltpu.SemaphoreType.DMA,
        ],
    )
    return run(x2, e2).reshape(_B, _L, _D)


# speedup vs baseline: 3.1415x; 2.7469x over previous
import jax
import jax.numpy as jnp
from jax import lax
from jax.experimental import pallas as pl
from jax.experimental.pallas import tpu as pltpu
from jax.experimental.pallas import tpu_sc as plsc

_B = 4
_L = 8192
_D = 1024
_NC = 2
_NS = 16
_NW = _NC * _NS
_RPW = _L // _NW
_C = 8
_NCH = _RPW // _C


def _sc_body(x_hbm, e_hbm, o_hbm, xb, eb, sin0, sin1, sout0, sout1):
    wid = lax.axis_index("s") * _NC + lax.axis_index("c")
    base = wid * _RPW
    sin = (sin0, sin1)
    sout = (sout0, sout1)

    def rows(c):
        return pl.ds(pl.multiple_of(base + c * _C, _C), _C)

    def fire_in(s, c):
        r = rows(c)
        pltpu.async_copy(e_hbm.at[r], eb.at[s], sin[s])
        for b in range(_B):
            pltpu.async_copy(x_hbm.at[b, r], xb.at[s, b], sin[s])

    def wait_in(s, c):
        r = rows(c)
        pltpu.make_async_copy(e_hbm.at[r], eb.at[s], sin[s]).wait()
        for b in range(_B):
            pltpu.make_async_copy(x_hbm.at[b, r], xb.at[s, b], sin[s]).wait()

    def fire_out(s, c):
        r = rows(c)
        for b in range(_B):
            pltpu.async_copy(xb.at[s, b], o_hbm.at[b, r], sout[s])

    def wait_out(s, c):
        r = rows(c)
        for b in range(_B):
            pltpu.make_async_copy(xb.at[s, b], o_hbm.at[b, r], sout[s]).wait()

    def compute(s):
        er = eb.at[s]
        xr = [xb.at[s, b] for b in range(_B)]

        @plsc.parallel_loop(0, _D, step=16, unroll=4)
        def step(i):
            sl = pl.ds(i, 16)
            for r in range(_C):
                v = er[r, sl]
                for b in range(_B):
                    xr[b][r, sl] = xr[b][r, sl] + v

    fire_in(0, 0)
    fire_in(1, 1)

    def body(k, carry):
        cc = 2 * k
        wait_in(0, cc)
        compute(0)
        fire_out(0, cc)
        wait_in(1, cc + 1)
        compute(1)
        fire_out(1, cc + 1)
        wait_out(0, cc)

        @pl.when(cc + 2 < _NCH)
        def _():
            fire_in(0, cc + 2)

        wait_out(1, cc + 1)

        @pl.when(cc + 3 < _NCH)
        def _():
            fire_in(1, cc + 3)

        return carry

    lax.fori_loop(0, _NCH // 2, body, 0)


def kernel(input, emb):
    run = pl.kernel(
        _sc_body,
        out_type=jax.ShapeDtypeStruct((_B, _L, _D), jnp.float32),
        mesh=plsc.VectorSubcoreMesh(core_axis_name="c", subcore_axis_name="s"),
        compiler_params=pltpu.CompilerParams(use_tc_tiling_on_sc=True),
        scratch_types=[
            pltpu.VMEM((2, _B, _C, _D), jnp.float32),
            pltpu.VMEM((2, _C, _D), jnp.float32),
            pltpu.SemaphoreType.DMA,
            pltpu.SemaphoreType.DMA,
            pltpu.SemaphoreType.DMA,
            pltpu.SemaphoreType.DMA,
        ],
    )
    return run(input, emb)
